# baseline (speedup 1.0000x reference)
import math
from functools import partial

import jax
import jax.numpy as jnp
from jax.experimental import pallas as pl
from jax.experimental.pallas import tpu as pltpu


def _add_instance_norm(x, res, w, b, *, batch, seq, eps):
    d = x.shape[-1]
    h = (x + res).reshape(batch, seq, d)
    mean = jnp.mean(h, axis=1, keepdims=True)
    c = h - mean
    var = jnp.mean(c * c, axis=1, keepdims=True)
    hn = c * jax.lax.rsqrt(var + eps)
    out = hn * w.reshape(1, 1, d) + b.reshape(1, 1, d)
    return out.reshape(batch * seq, d)


def _encoder_kernel(feats_ref, wcomb_ref,
                    wqkv_ref, bqkv_ref, wo_ref,
                    w1_ref, b1_ref, w2_ref,
                    n1w_ref, n1b_ref, n2w_ref, n2b_ref,
                    init_ref, h_ref, kbd_ref, vbd_ref,
                    *, num_layers, batch, seq, num_heads, eps):
    M, D = init_ref.shape
    H = num_heads
    hd = D // H
    scale = 1.0 / math.sqrt(hd)
    nt = (((1,), (1,)), ((), ()))

    x = jnp.dot(feats_ref[...], wcomb_ref[...],
                preferred_element_type=jnp.float32)
    init_ref[...] = x

    kbd_ref[...] = jnp.zeros_like(kbd_ref)
    vbd_ref[...] = jnp.zeros_like(vbd_ref)
    nbuf = kbd_ref.shape[0]

    for l in range(num_layers):
        xb = x.astype(jnp.bfloat16)
        wqkv_b = wqkv_ref[l].astype(jnp.bfloat16)

        qkv = jnp.dot(xb, wqkv_b, preferred_element_type=jnp.float32)
        qs = ((qkv[:, 0:D] + bqkv_ref[l, :, 0:D]) * scale).astype(jnp.bfloat16)
        kv = qkv[:, D:3 * D].astype(jnp.bfloat16)

        o_rows = []
        for bi in range(batch):
            r0 = bi * seq
            pb = bi % nbuf
            for hh in range(H):
                c = hh * hd
                kbd_ref[pb, hh * seq:(hh + 1) * seq, c:c + hd] = \
                    kv[r0:r0 + seq, c:c + hd]
                vbd_ref[pb, hh * seq:(hh + 1) * seq, c:c + hd] = \
                    kv[r0:r0 + seq, D + c:D + c + hd]
            q = qs[r0:r0 + seq, :]
            s_cat = jax.lax.dot_general(
                q, kbd_ref[pb], nt, preferred_element_type=jnp.float32)
            ps = []
            for hh in range(H):
                s = s_cat[:, hh * seq:(hh + 1) * seq]
                p = jnp.exp(jnp.minimum(s, 80.0))
                p = p * pl.reciprocal(jnp.sum(p, axis=-1, keepdims=True),
                                      approx=True)
                ps.append(p.astype(jnp.bfloat16))
            p_cat = jnp.concatenate(ps, axis=1)
            o_rows.append(jnp.dot(p_cat, vbd_ref[pb],
                                  preferred_element_type=jnp.float32))
        o_all = jnp.concatenate(o_rows, axis=0)
        attn_out = jnp.dot(o_all.astype(jnp.bfloat16),
                           wo_ref[l].astype(jnp.bfloat16),
                           preferred_element_type=jnp.float32)

        h1 = _add_instance_norm(attn_out, x, n1w_ref[l], n1b_ref[l],
                                batch=batch, seq=seq, eps=eps)

        f = jnp.dot(h1.astype(jnp.bfloat16), w1_ref[l].astype(jnp.bfloat16),
                    preferred_element_type=jnp.float32) + b1_ref[l]
        f = jnp.maximum(f, 0.0)
        ffn_out = jnp.dot(f.astype(jnp.bfloat16),
                          w2_ref[l].astype(jnp.bfloat16),
                          preferred_element_type=jnp.float32)
        x = _add_instance_norm(ffn_out, h1, n2w_ref[l], n2b_ref[l],
                               batch=batch, seq=seq, eps=eps)

    h_ref[...] = x


def kernel(depot_feats, node_feats, wqkv, bqkv, wo, bo, w1, b1, w2, b2,
           depot_w, node_w, n1_w, n1_b, n2_w, n2_b):
    B, _, Fd = depot_feats.shape
    _, Nc, Fn = node_feats.shape
    D = depot_w.shape[1]
    N = Nc + 1
    M = B * N
    L = wqkv.shape[0]
    H = 8
    eps = 1e-5

    depot_pad = jnp.pad(depot_feats, ((0, 0), (0, 0), (0, Fn)))
    node_pad = jnp.pad(node_feats, ((0, 0), (0, 0), (Fd, 0)))
    feats = jnp.concatenate([depot_pad, node_pad], axis=1).reshape(M, Fd + Fn)
    wcomb = jnp.concatenate([depot_w, node_w], axis=0)

    F = w1.shape[2]

    def full(shape):
        nd = len(shape)
        return pl.BlockSpec(shape, lambda: (0,) * nd)

    body = partial(_encoder_kernel, num_layers=L, batch=B, seq=N,
                   num_heads=H, eps=eps)
    init_h, h_out = pl.pallas_call(
        body,
        out_shape=(jax.ShapeDtypeStruct((M, D), jnp.float32),
                   jax.ShapeDtypeStruct((M, D), jnp.float32)),
        in_specs=[
            full((M, Fd + Fn)),
            full((Fd + Fn, D)),
            full((L, D, 3 * D)), full((L, 1, 3 * D)),
            full((L, D, D)),
            full((L, D, F)), full((L, 1, F)),
            full((L, F, D)),
            full((L, 1, D)), full((L, 1, D)),
            full((L, 1, D)), full((L, 1, D)),
        ],
        out_specs=(full((M, D)), full((M, D))),
        scratch_shapes=[pltpu.VMEM((4, H * N, D), jnp.bfloat16),
                        pltpu.VMEM((4, H * N, D), jnp.bfloat16)],
        compiler_params=pltpu.CompilerParams(
            vmem_limit_bytes=100 * 1024 * 1024),
    )(feats, wcomb,
      wqkv, bqkv, wo,
      w1, b1, w2,
      n1_w, n1_b, n2_w, n2_b)

    return h_out.reshape(B, N, D), init_h.reshape(B, N, D)


# speedup vs baseline: 1.1004x; 1.1004x over previous
import math
from functools import partial

import jax
import jax.numpy as jnp
from jax.experimental import pallas as pl
from jax.experimental.pallas import tpu as pltpu


def _add_instance_norm(x, res, w, b, *, batch, seq, eps):
    d = x.shape[-1]
    h = (x + res).reshape(batch, seq, d)
    mean = jnp.mean(h, axis=1, keepdims=True)
    c = h - mean
    var = jnp.mean(c * c, axis=1, keepdims=True)
    hn = c * jax.lax.rsqrt(var + eps)
    out = hn * w.reshape(1, 1, d) + b.reshape(1, 1, d)
    return out.reshape(batch * seq, d)


def _encoder_kernel(feats_ref, wcomb_ref,
                    wqkv_ref, bqkv_ref, wo_ref, bo_ref,
                    w1_ref, b1_ref, w2_ref, b2_ref,
                    n1w_ref, n1b_ref, n2w_ref, n2b_ref,
                    init_ref, h_ref, kbd_ref, vbd_ref,
                    *, batch, seq, num_heads, eps):
    @pl.when(pl.program_id(0) == 0)
    def _():
        ih = jnp.dot(feats_ref[...], wcomb_ref[...],
                     preferred_element_type=jnp.float32)
        init_ref[...] = ih
        h_ref[...] = ih
        kbd_ref[...] = jnp.zeros_like(kbd_ref)
        vbd_ref[...] = jnp.zeros_like(vbd_ref)

    nbuf = kbd_ref.shape[0]

    _, D = h_ref.shape
    H = num_heads
    hd = D // H
    scale = 1.0 / math.sqrt(hd)
    nt = (((1,), (1,)), ((), ()))

    x = h_ref[...]

    xb = x.astype(jnp.bfloat16)
    wqkv_b = wqkv_ref[0].astype(jnp.bfloat16)

    qkv = jnp.dot(xb, wqkv_b, preferred_element_type=jnp.float32)
    qs = (qkv[:, 0:D] + bqkv_ref[0, :, 0:D]).astype(jnp.bfloat16)

    o_rows = []
    for bi in range(batch):
        r0 = bi * seq
        pb = bi % nbuf
        for hh in range(H):
            c = hh * hd
            kbd_ref[pb, hh * seq:(hh + 1) * seq, c:c + hd] = \
                qkv[r0:r0 + seq, D + c:D + c + hd].astype(jnp.bfloat16)
            vbd_ref[pb, hh * seq:(hh + 1) * seq, c:c + hd] = \
                qkv[r0:r0 + seq, 2 * D + c:2 * D + c + hd].astype(jnp.bfloat16)
        q = qs[r0:r0 + seq, :]
        s_cat = jax.lax.dot_general(
            q, kbd_ref[pb], nt, preferred_element_type=jnp.float32)
        ps = []
        for hh in range(H):
            s = s_cat[:, hh * seq:(hh + 1) * seq]
            p = jnp.exp2(jnp.minimum(s * (scale * 1.4426950408889634), 115.0))
            p = p * pl.reciprocal(jnp.sum(p, axis=-1, keepdims=True),
                                  approx=True)
            ps.append(p.astype(jnp.bfloat16))
        p_cat = jnp.concatenate(ps, axis=1)
        o_rows.append(jnp.dot(p_cat, vbd_ref[pb],
                              preferred_element_type=jnp.float32))
    o_all = jnp.concatenate(o_rows, axis=0)
    attn_out = jnp.dot(o_all.astype(jnp.bfloat16),
                       wo_ref[0].astype(jnp.bfloat16),
                       preferred_element_type=jnp.float32)

    h1 = _add_instance_norm(attn_out, x, n1w_ref[0], n1b_ref[0],
                            batch=batch, seq=seq, eps=eps)

    f = jnp.dot(h1.astype(jnp.bfloat16), w1_ref[0].astype(jnp.bfloat16),
                preferred_element_type=jnp.float32) + b1_ref[0]
    f = jnp.maximum(f, 0.0)
    ffn_out = jnp.dot(f.astype(jnp.bfloat16), w2_ref[0].astype(jnp.bfloat16),
                      preferred_element_type=jnp.float32)
    h2 = _add_instance_norm(ffn_out, h1, n2w_ref[0], n2b_ref[0],
                            batch=batch, seq=seq, eps=eps)

    h_ref[...] = h2


def kernel(depot_feats, node_feats, wqkv, bqkv, wo, bo, w1, b1, w2, b2,
           depot_w, node_w, n1_w, n1_b, n2_w, n2_b):
    B, _, Fd = depot_feats.shape
    _, Nc, Fn = node_feats.shape
    D = depot_w.shape[1]
    N = Nc + 1
    M = B * N
    L = wqkv.shape[0]
    H = 8
    eps = 1e-5

    depot_pad = jnp.pad(depot_feats, ((0, 0), (0, 0), (0, Fn)))
    node_pad = jnp.pad(node_feats, ((0, 0), (0, 0), (Fd, 0)))
    feats = jnp.concatenate([depot_pad, node_pad], axis=1).reshape(M, Fd + Fn)
    wcomb = jnp.concatenate([depot_w, node_w], axis=0)

    F = w1.shape[2]

    def full2d(shape):
        return pl.BlockSpec(shape, lambda l: (0, 0))

    def per_layer(shape):
        return pl.BlockSpec((1,) + shape, lambda l: (l, 0, 0))

    body = partial(_encoder_kernel, batch=B, seq=N, num_heads=H, eps=eps)
    init_h, h_out = pl.pallas_call(
        body,
        out_shape=(jax.ShapeDtypeStruct((M, D), jnp.float32),
                   jax.ShapeDtypeStruct((M, D), jnp.float32)),
        grid=(L,),
        in_specs=[
            full2d((M, Fd + Fn)),
            full2d((Fd + Fn, D)),
            per_layer((D, 3 * D)), per_layer((1, 3 * D)),
            per_layer((D, D)), per_layer((1, D)),
            per_layer((D, F)), per_layer((1, F)),
            per_layer((F, D)), per_layer((1, D)),
            per_layer((1, D)), per_layer((1, D)),
            per_layer((1, D)), per_layer((1, D)),
        ],
        out_specs=(full2d((M, D)), full2d((M, D))),
        scratch_shapes=[pltpu.VMEM((4, H * N, D), jnp.bfloat16),
                        pltpu.VMEM((4, H * N, D), jnp.bfloat16)],
        compiler_params=pltpu.CompilerParams(
            dimension_semantics=("arbitrary",)),
    )(feats, wcomb,
      wqkv, bqkv, wo, bo,
      w1, b1, w2, b2,
      n1_w, n1_b, n2_w, n2_b)

    return h_out.reshape(B, N, D), init_h.reshape(B, N, D)


# speedup vs baseline: 1.1022x; 1.0017x over previous
import math
from functools import partial

import jax
import jax.numpy as jnp
from jax.experimental import pallas as pl
from jax.experimental.pallas import tpu as pltpu


def _add_instance_norm(x, res, w, b, *, batch, seq, eps):
    d = x.shape[-1]
    h = (x + res).reshape(batch, seq, d)
    mean = jnp.mean(h, axis=1, keepdims=True)
    c = h - mean
    var = jnp.mean(c * c, axis=1, keepdims=True)
    hn = c * jax.lax.rsqrt(var + eps)
    out = hn * w.reshape(1, 1, d) + b.reshape(1, 1, d)
    return out.reshape(batch * seq, d)


def _encoder_kernel(feats_ref, wcomb_ref,
                    wqkv_ref, bqkv_ref, wo_ref, bo_ref,
                    w1_ref, b1_ref, w2_ref, b2_ref,
                    n1w_ref, n1b_ref, n2w_ref, n2b_ref,
                    init_ref, h_ref, kbd_ref, vbd_ref,
                    *, batch, seq, num_heads, eps):
    @pl.when(pl.program_id(0) == 0)
    def _():
        ih = jnp.dot(feats_ref[...], wcomb_ref[...],
                     preferred_element_type=jnp.float32)
        init_ref[...] = ih
        h_ref[...] = ih
        kbd_ref[...] = jnp.zeros_like(kbd_ref)
        vbd_ref[...] = jnp.zeros_like(vbd_ref)

    nbuf = kbd_ref.shape[0]

    _, D = h_ref.shape
    H = num_heads
    hd = D // H
    scale = 1.0 / math.sqrt(hd)
    nt = (((1,), (1,)), ((), ()))

    x = h_ref[...]

    xb = x.astype(jnp.bfloat16)
    wqkv_b = wqkv_ref[0].astype(jnp.bfloat16)

    qs = (jnp.dot(xb, wqkv_b[:, 0:D], preferred_element_type=jnp.float32)
          + bqkv_ref[0, :, 0:D]).astype(jnp.bfloat16)
    qkv = jnp.dot(xb, wqkv_b[:, D:3 * D], preferred_element_type=jnp.float32)

    o_rows = []
    for bi in range(batch):
        r0 = bi * seq
        pb = bi % nbuf
        for hh in range(H):
            c = hh * hd
            kbd_ref[pb, hh * seq:(hh + 1) * seq, c:c + hd] = \
                qkv[r0:r0 + seq, c:c + hd].astype(jnp.bfloat16)
            vbd_ref[pb, hh * seq:(hh + 1) * seq, c:c + hd] = \
                qkv[r0:r0 + seq, D + c:D + c + hd].astype(jnp.bfloat16)
        q = qs[r0:r0 + seq, :]
        s_cat = jax.lax.dot_general(
            q, kbd_ref[pb], nt, preferred_element_type=jnp.float32)
        ps = []
        for hh in range(H):
            s = s_cat[:, hh * seq:(hh + 1) * seq]
            p = jnp.exp2(jnp.minimum(s * (scale * 1.4426950408889634), 115.0))
            p = p * pl.reciprocal(jnp.sum(p, axis=-1, keepdims=True),
                                  approx=True)
            ps.append(p.astype(jnp.bfloat16))
        p_cat = jnp.concatenate(ps, axis=1)
        o_rows.append(jnp.dot(p_cat, vbd_ref[pb],
                              preferred_element_type=jnp.float32))
    o_all = jnp.concatenate(o_rows, axis=0)
    attn_out = jnp.dot(o_all.astype(jnp.bfloat16),
                       wo_ref[0].astype(jnp.bfloat16),
                       preferred_element_type=jnp.float32)

    h1 = _add_instance_norm(attn_out, x, n1w_ref[0], n1b_ref[0],
                            batch=batch, seq=seq, eps=eps)

    h1b = h1.astype(jnp.bfloat16)
    w1b = w1_ref[0].astype(jnp.bfloat16)
    Fh = w1b.shape[1] // 2
    fs = []
    for ci in range(2):
        fc = jnp.dot(h1b, w1b[:, ci * Fh:(ci + 1) * Fh],
                     preferred_element_type=jnp.float32) \
             + b1_ref[0, :, ci * Fh:(ci + 1) * Fh]
        fs.append(jnp.maximum(fc, 0.0).astype(jnp.bfloat16))
    f = jnp.concatenate(fs, axis=1)
    ffn_out = jnp.dot(f, w2_ref[0].astype(jnp.bfloat16),
                      preferred_element_type=jnp.float32)
    h2 = _add_instance_norm(ffn_out, h1, n2w_ref[0], n2b_ref[0],
                            batch=batch, seq=seq, eps=eps)

    h_ref[...] = h2


def kernel(depot_feats, node_feats, wqkv, bqkv, wo, bo, w1, b1, w2, b2,
           depot_w, node_w, n1_w, n1_b, n2_w, n2_b):
    B, _, Fd = depot_feats.shape
    _, Nc, Fn = node_feats.shape
    D = depot_w.shape[1]
    N = Nc + 1
    M = B * N
    L = wqkv.shape[0]
    H = 8
    eps = 1e-5

    depot_pad = jnp.pad(depot_feats, ((0, 0), (0, 0), (0, Fn)))
    node_pad = jnp.pad(node_feats, ((0, 0), (0, 0), (Fd, 0)))
    feats = jnp.concatenate([depot_pad, node_pad], axis=1).reshape(M, Fd + Fn)
    wcomb = jnp.concatenate([depot_w, node_w], axis=0)

    F = w1.shape[2]

    def full2d(shape):
        return pl.BlockSpec(shape, lambda l: (0, 0))

    def per_layer(shape):
        return pl.BlockSpec((1,) + shape, lambda l: (l, 0, 0))

    body = partial(_encoder_kernel, batch=B, seq=N, num_heads=H, eps=eps)
    init_h, h_out = pl.pallas_call(
        body,
        out_shape=(jax.ShapeDtypeStruct((M, D), jnp.float32),
                   jax.ShapeDtypeStruct((M, D), jnp.float32)),
        grid=(L,),
        in_specs=[
            full2d((M, Fd + Fn)),
            full2d((Fd + Fn, D)),
            per_layer((D, 3 * D)), per_layer((1, 3 * D)),
            per_layer((D, D)), per_layer((1, D)),
            per_layer((D, F)), per_layer((1, F)),
            per_layer((F, D)), per_layer((1, D)),
            per_layer((1, D)), per_layer((1, D)),
            per_layer((1, D)), per_layer((1, D)),
        ],
        out_specs=(full2d((M, D)), full2d((M, D))),
        scratch_shapes=[pltpu.VMEM((4, H * N, D), jnp.bfloat16),
                        pltpu.VMEM((4, H * N, D), jnp.bfloat16)],
        compiler_params=pltpu.CompilerParams(
            dimension_semantics=("arbitrary",)),
    )(feats, wcomb,
      wqkv, bqkv, wo, bo,
      w1, b1, w2, b2,
      n1_w, n1_b, n2_w, n2_b)

    return h_out.reshape(B, N, D), init_h.reshape(B, N, D)


# speedup vs baseline: 1.1392x; 1.0336x over previous
import math
from functools import partial

import jax
import jax.numpy as jnp
from jax.experimental import pallas as pl
from jax.experimental.pallas import tpu as pltpu


def _add_instance_norm(x, res, w, b, *, batch, seq, eps):
    d = x.shape[-1]
    h = (x + res).reshape(batch, seq, d)
    mean = jnp.mean(h, axis=1, keepdims=True)
    c = h - mean
    var = jnp.mean(c * c, axis=1, keepdims=True)
    hn = c * jax.lax.rsqrt(var + eps)
    out = hn * w.reshape(1, 1, d) + b.reshape(1, 1, d)
    return out.reshape(batch * seq, d)


def _encoder_kernel(feats_ref, wcomb_ref,
                    wqkv_ref, bqkv_ref, wo_ref, bo_ref,
                    w1_ref, b1_ref, w2_ref, b2_ref,
                    n1w_ref, n1b_ref, n2w_ref, n2b_ref,
                    init_ref, h_ref, kbd_ref, vbd_ref, ones_ref,
                    *, batch, seq, num_heads, eps):
    @pl.when(pl.program_id(0) == 0)
    def _():
        ih = jnp.dot(feats_ref[...], wcomb_ref[...],
                     preferred_element_type=jnp.float32)
        init_ref[...] = ih
        h_ref[...] = ih
        kbd_ref[...] = jnp.zeros_like(kbd_ref)
        vbd_ref[...] = jnp.zeros_like(vbd_ref)
        ii = jax.lax.broadcasted_iota(jnp.int32, ones_ref.shape, 0)
        jj = jax.lax.broadcasted_iota(jnp.int32, ones_ref.shape, 1)
        ones_ref[...] = jnp.where(ii // seq == jj, 1.0, 0.0).astype(ones_ref.dtype)

    nbuf = kbd_ref.shape[0]

    _, D = h_ref.shape
    H = num_heads
    hd = D // H
    scale = 1.0 / math.sqrt(hd)
    nt = (((1,), (1,)), ((), ()))

    x = h_ref[...]

    xb = x.astype(jnp.bfloat16)
    wqkv_b = wqkv_ref[0].astype(jnp.bfloat16)

    qs = (jnp.dot(xb, wqkv_b[:, 0:D], preferred_element_type=jnp.float32)
          + bqkv_ref[0, :, 0:D]).astype(jnp.bfloat16)
    qkv = jnp.dot(xb, wqkv_b[:, D:3 * D], preferred_element_type=jnp.float32)

    hh_i = jax.lax.broadcasted_iota(jnp.int32, (H, D), 0)
    cc_i = jax.lax.broadcasted_iota(jnp.int32, (H, D), 1)
    sel = jnp.where(cc_i // hd == hh_i, 1.0, 0.0)

    o_rows = []
    for bi in range(batch):
        r0 = bi * seq
        pb = bi % nbuf
        for hh in range(H):
            c = hh * hd
            kbd_ref[pb, hh * seq:(hh + 1) * seq, c:c + hd] = \
                qkv[r0:r0 + seq, c:c + hd].astype(jnp.bfloat16)
            vbd_ref[pb, hh * seq:(hh + 1) * seq, c:c + hd] = \
                qkv[r0:r0 + seq, D + c:D + c + hd].astype(jnp.bfloat16)
        q = qs[r0:r0 + seq, :]
        s_cat = jax.lax.dot_general(
            q, kbd_ref[pb], nt, preferred_element_type=jnp.float32)
        p_cat = jnp.exp2(jnp.minimum(
            s_cat * (scale * 1.4426950408889634), 100.0)).astype(jnp.bfloat16)
        sums = jnp.dot(p_cat, ones_ref[...],
                       preferred_element_type=jnp.float32)
        o_uc = jnp.dot(p_cat, vbd_ref[pb],
                       preferred_element_type=jnp.float32)
        rec = pl.reciprocal(sums, approx=True)
        o_rows.append(o_uc * jnp.dot(rec, sel, preferred_element_type=jnp.float32))
    o_all = jnp.concatenate(o_rows, axis=0)
    attn_out = jnp.dot(o_all.astype(jnp.bfloat16),
                       wo_ref[0].astype(jnp.bfloat16),
                       preferred_element_type=jnp.float32)

    h1 = _add_instance_norm(attn_out, x, n1w_ref[0], n1b_ref[0],
                            batch=batch, seq=seq, eps=eps)

    h1b = h1.astype(jnp.bfloat16)
    w1b = w1_ref[0].astype(jnp.bfloat16)
    Fh = w1b.shape[1] // 2
    fs = []
    for ci in range(2):
        fc = jnp.dot(h1b, w1b[:, ci * Fh:(ci + 1) * Fh],
                     preferred_element_type=jnp.float32) \
             + b1_ref[0, :, ci * Fh:(ci + 1) * Fh]
        fs.append(jnp.maximum(fc, 0.0).astype(jnp.bfloat16))
    f = jnp.concatenate(fs, axis=1)
    ffn_out = jnp.dot(f, w2_ref[0].astype(jnp.bfloat16),
                      preferred_element_type=jnp.float32)
    h2 = _add_instance_norm(ffn_out, h1, n2w_ref[0], n2b_ref[0],
                            batch=batch, seq=seq, eps=eps)

    h_ref[...] = h2


def kernel(depot_feats, node_feats, wqkv, bqkv, wo, bo, w1, b1, w2, b2,
           depot_w, node_w, n1_w, n1_b, n2_w, n2_b):
    B, _, Fd = depot_feats.shape
    _, Nc, Fn = node_feats.shape
    D = depot_w.shape[1]
    N = Nc + 1
    M = B * N
    L = wqkv.shape[0]
    H = 8
    eps = 1e-5

    depot_pad = jnp.pad(depot_feats, ((0, 0), (0, 0), (0, Fn)))
    node_pad = jnp.pad(node_feats, ((0, 0), (0, 0), (Fd, 0)))
    feats = jnp.concatenate([depot_pad, node_pad], axis=1).reshape(M, Fd + Fn)
    wcomb = jnp.concatenate([depot_w, node_w], axis=0)

    F = w1.shape[2]

    def full2d(shape):
        return pl.BlockSpec(shape, lambda l: (0, 0))

    def per_layer(shape):
        return pl.BlockSpec((1,) + shape, lambda l: (l, 0, 0))

    body = partial(_encoder_kernel, batch=B, seq=N, num_heads=H, eps=eps)
    init_h, h_out = pl.pallas_call(
        body,
        out_shape=(jax.ShapeDtypeStruct((M, D), jnp.float32),
                   jax.ShapeDtypeStruct((M, D), jnp.float32)),
        grid=(L,),
        in_specs=[
            full2d((M, Fd + Fn)),
            full2d((Fd + Fn, D)),
            per_layer((D, 3 * D)), per_layer((1, 3 * D)),
            per_layer((D, D)), per_layer((1, D)),
            per_layer((D, F)), per_layer((1, F)),
            per_layer((F, D)), per_layer((1, D)),
            per_layer((1, D)), per_layer((1, D)),
            per_layer((1, D)), per_layer((1, D)),
        ],
        out_specs=(full2d((M, D)), full2d((M, D))),
        scratch_shapes=[pltpu.VMEM((4, H * N, D), jnp.bfloat16),
                        pltpu.VMEM((4, H * N, D), jnp.bfloat16),
                        pltpu.VMEM((H * N, H), jnp.bfloat16)],
        compiler_params=pltpu.CompilerParams(
            dimension_semantics=("arbitrary",)),
    )(feats, wcomb,
      wqkv, bqkv, wo, bo,
      w1, b1, w2, b2,
      n1_w, n1_b, n2_w, n2_b)

    return h_out.reshape(B, N, D), init_h.reshape(B, N, D)


# speedup vs baseline: 1.1549x; 1.0138x over previous
import math
from functools import partial

import jax
import jax.numpy as jnp
from jax.experimental import pallas as pl
from jax.experimental.pallas import tpu as pltpu


def _add_instance_norm(x, res, w, b, *, batch, seq, eps):
    d = x.shape[-1]
    h = (x + res).reshape(batch, seq, d)
    mean = jnp.mean(h, axis=1, keepdims=True)
    c = h - mean
    var = jnp.mean(c * c, axis=1, keepdims=True)
    hn = c * jax.lax.rsqrt(var + eps)
    out = hn * w.reshape(1, 1, d) + b.reshape(1, 1, d)
    return out.reshape(batch * seq, d)


def _encoder_kernel(feats_ref, wcomb_ref,
                    wqkv_ref, bqkv_ref, wo_ref, bo_ref,
                    w1_ref, b1_ref, w2_ref, b2_ref,
                    n1w_ref, n1b_ref, n2w_ref, n2b_ref,
                    init_ref, h_ref, kbd_ref, vbd_ref, ones_ref,
                    *, batch, seq, num_heads, eps):
    @pl.when(pl.program_id(0) == 0)
    def _():
        ih = jnp.dot(feats_ref[...], wcomb_ref[...],
                     preferred_element_type=jnp.float32)
        init_ref[...] = ih
        h_ref[...] = ih
        kbd_ref[...] = jnp.zeros_like(kbd_ref)
        vbd_ref[...] = jnp.zeros_like(vbd_ref)
        ii = jax.lax.broadcasted_iota(jnp.int32, ones_ref.shape, 0)
        jj = jax.lax.broadcasted_iota(jnp.int32, ones_ref.shape, 1)
        ones_ref[...] = jnp.where(ii // seq == jj, 1.0, 0.0).astype(ones_ref.dtype)

    nbuf = kbd_ref.shape[0]

    _, D = h_ref.shape
    H = num_heads
    hd = D // H
    scale = 1.0 / math.sqrt(hd)
    nt = (((1,), (1,)), ((), ()))

    x = h_ref[...]

    xb = x.astype(jnp.bfloat16)
    wqkv_b = wqkv_ref[0].astype(jnp.bfloat16)

    qs = jnp.dot(xb, wqkv_b[:, 0:D],
                 preferred_element_type=jnp.float32).astype(jnp.bfloat16) \
         + bqkv_ref[0, :, 0:D].astype(jnp.bfloat16)
    qkv = jnp.dot(xb, wqkv_b[:, D:3 * D], preferred_element_type=jnp.float32)

    hh_i = jax.lax.broadcasted_iota(jnp.int32, (H, D), 0)
    cc_i = jax.lax.broadcasted_iota(jnp.int32, (H, D), 1)
    sel = jnp.where(cc_i // hd == hh_i, 1.0, 0.0)

    o_rows = []
    for bi in range(batch):
        r0 = bi * seq
        pb = bi % nbuf
        for hh in range(H):
            c = hh * hd
            kbd_ref[pb, hh * seq:(hh + 1) * seq, c:c + hd] = \
                qkv[r0:r0 + seq, c:c + hd].astype(jnp.bfloat16)
            vbd_ref[pb, hh * seq:(hh + 1) * seq, c:c + hd] = \
                qkv[r0:r0 + seq, D + c:D + c + hd].astype(jnp.bfloat16)
        q = qs[r0:r0 + seq, :]
        s_cat = jax.lax.dot_general(
            q, kbd_ref[pb], nt, preferred_element_type=jnp.float32)
        p_cat = jnp.exp2(jnp.minimum(
            s_cat * (scale * 1.4426950408889634), 100.0)).astype(jnp.bfloat16)
        sums = jnp.dot(p_cat, ones_ref[...],
                       preferred_element_type=jnp.float32)
        o_uc = jnp.dot(p_cat, vbd_ref[pb],
                       preferred_element_type=jnp.float32)
        rec = pl.reciprocal(sums, approx=True)
        o_rows.append(o_uc * jnp.dot(rec, sel, preferred_element_type=jnp.float32))
    o_all = jnp.concatenate(o_rows, axis=0)
    attn_out = jnp.dot(o_all.astype(jnp.bfloat16),
                       wo_ref[0].astype(jnp.bfloat16),
                       preferred_element_type=jnp.float32)

    h1 = _add_instance_norm(attn_out, x, n1w_ref[0], n1b_ref[0],
                            batch=batch, seq=seq, eps=eps)

    h1b = h1.astype(jnp.bfloat16)
    w1b = w1_ref[0].astype(jnp.bfloat16)
    Fh = w1b.shape[1] // 2
    fs = []
    for ci in range(2):
        fc = jnp.dot(h1b, w1b[:, ci * Fh:(ci + 1) * Fh],
                     preferred_element_type=jnp.float32).astype(jnp.bfloat16) \
             + b1_ref[0, :, ci * Fh:(ci + 1) * Fh].astype(jnp.bfloat16)
        fs.append(jnp.maximum(fc, jnp.bfloat16(0.0)))
    f = jnp.concatenate(fs, axis=1)
    ffn_out = jnp.dot(f, w2_ref[0].astype(jnp.bfloat16),
                      preferred_element_type=jnp.float32)
    h2 = _add_instance_norm(ffn_out, h1, n2w_ref[0], n2b_ref[0],
                            batch=batch, seq=seq, eps=eps)

    h_ref[...] = h2


def kernel(depot_feats, node_feats, wqkv, bqkv, wo, bo, w1, b1, w2, b2,
           depot_w, node_w, n1_w, n1_b, n2_w, n2_b):
    B, _, Fd = depot_feats.shape
    _, Nc, Fn = node_feats.shape
    D = depot_w.shape[1]
    N = Nc + 1
    M = B * N
    L = wqkv.shape[0]
    H = 8
    eps = 1e-5

    depot_pad = jnp.pad(depot_feats, ((0, 0), (0, 0), (0, Fn)))
    node_pad = jnp.pad(node_feats, ((0, 0), (0, 0), (Fd, 0)))
    feats = jnp.concatenate([depot_pad, node_pad], axis=1).reshape(M, Fd + Fn)
    wcomb = jnp.concatenate([depot_w, node_w], axis=0)

    F = w1.shape[2]

    def full2d(shape):
        return pl.BlockSpec(shape, lambda l: (0, 0))

    def per_layer(shape):
        return pl.BlockSpec((1,) + shape, lambda l: (l, 0, 0))

    body = partial(_encoder_kernel, batch=B, seq=N, num_heads=H, eps=eps)
    init_h, h_out = pl.pallas_call(
        body,
        out_shape=(jax.ShapeDtypeStruct((M, D), jnp.float32),
                   jax.ShapeDtypeStruct((M, D), jnp.float32)),
        grid=(L,),
        in_specs=[
            full2d((M, Fd + Fn)),
            full2d((Fd + Fn, D)),
            per_layer((D, 3 * D)), per_layer((1, 3 * D)),
            per_layer((D, D)), per_layer((1, D)),
            per_layer((D, F)), per_layer((1, F)),
            per_layer((F, D)), per_layer((1, D)),
            per_layer((1, D)), per_layer((1, D)),
            per_layer((1, D)), per_layer((1, D)),
        ],
        out_specs=(full2d((M, D)), full2d((M, D))),
        scratch_shapes=[pltpu.VMEM((8, H * N, D), jnp.bfloat16),
                        pltpu.VMEM((8, H * N, D), jnp.bfloat16),
                        pltpu.VMEM((H * N, H), jnp.bfloat16)],
        compiler_params=pltpu.CompilerParams(
            dimension_semantics=("arbitrary",)),
    )(feats, wcomb,
      wqkv, bqkv, wo, bo,
      w1, b1, w2, b2,
      n1_w, n1_b, n2_w, n2_b)

    return h_out.reshape(B, N, D), init_h.reshape(B, N, D)


# speedup vs baseline: 1.1818x; 1.0233x over previous
import math
from functools import partial

import jax
import jax.numpy as jnp
from jax.experimental import pallas as pl
from jax.experimental.pallas import tpu as pltpu


def _add_instance_norm(x, res, w, b, *, batch, seq, eps):
    d = x.shape[-1]
    h = (x + res).reshape(batch, seq, d)
    mean = jnp.mean(h, axis=1, keepdims=True)
    c = h - mean
    var = jnp.mean(c * c, axis=1, keepdims=True)
    hn = c * jax.lax.rsqrt(var + eps)
    out = hn * w.reshape(1, 1, d) + b.reshape(1, 1, d)
    return out.reshape(batch * seq, d)


def _encoder_kernel(feats_ref, wcomb_ref,
                    wqkv_ref, bqkv_ref, wo_ref, bo_ref,
                    w1_ref, b1_ref, w2_ref, b2_ref,
                    n1w_ref, n1b_ref, n2w_ref, n2b_ref,
                    init_ref, h_ref, kbd_ref, vbd_ref,
                    *, batch, seq, num_heads, eps):
    @pl.when(pl.program_id(0) == 0)
    def _():
        ih = jnp.dot(feats_ref[...], wcomb_ref[...],
                     preferred_element_type=jnp.float32)
        init_ref[...] = ih
        h_ref[...] = ih
        kbd_ref[...] = jnp.zeros_like(kbd_ref)
        nb, rows, _ = vbd_ref.shape
        ii = jax.lax.broadcasted_iota(jnp.int32, (nb, rows, num_heads), 1)
        jj = jax.lax.broadcasted_iota(jnp.int32, (nb, rows, num_heads), 2)
        vbd_ref[:, :, 0:0 + vbd_ref.shape[2]] = jnp.zeros_like(vbd_ref)
        vbd_ref[:, :, kbd_ref.shape[2]:] = \
            jnp.where(ii // seq == jj, 1.0, 0.0).astype(vbd_ref.dtype)

    nbuf = kbd_ref.shape[0]

    _, D = h_ref.shape
    H = num_heads
    hd = D // H
    scale = 1.0 / math.sqrt(hd)
    nt = (((1,), (1,)), ((), ()))

    x = h_ref[...]

    xb = x.astype(jnp.bfloat16)
    wqkv_b = wqkv_ref[0].astype(jnp.bfloat16)

    qs = jnp.dot(xb, wqkv_b[:, 0:D],
                 preferred_element_type=jnp.float32).astype(jnp.bfloat16) \
         + bqkv_ref[0, :, 0:D].astype(jnp.bfloat16)
    qkv = jnp.dot(xb, wqkv_b[:, D:3 * D], preferred_element_type=jnp.float32)

    hh_i = jax.lax.broadcasted_iota(jnp.int32, (H, D), 0)
    cc_i = jax.lax.broadcasted_iota(jnp.int32, (H, D), 1)
    sel = jnp.where(cc_i // hd == hh_i, 1.0, 0.0)

    o_rows = []
    for bi in range(batch):
        r0 = bi * seq
        pb = bi % nbuf
        for hh in range(H):
            c = hh * hd
            kbd_ref[pb, hh * seq:(hh + 1) * seq, c:c + hd] = \
                qkv[r0:r0 + seq, c:c + hd].astype(jnp.bfloat16)
            vbd_ref[pb, hh * seq:(hh + 1) * seq, c:c + hd] = \
                qkv[r0:r0 + seq, D + c:D + c + hd].astype(jnp.bfloat16)
        q = qs[r0:r0 + seq, :]
        s_cat = jax.lax.dot_general(
            q, kbd_ref[pb], nt, preferred_element_type=jnp.float32)
        p_cat = jnp.exp2(jnp.minimum(
            s_cat * (scale * 1.4426950408889634), 100.0)).astype(jnp.bfloat16)
        o_ext = jnp.dot(p_cat, vbd_ref[pb],
                        preferred_element_type=jnp.float32)
        o_rows.append(o_ext)
    o_all_ext = jnp.concatenate(o_rows, axis=0)
    rec = pl.reciprocal(o_all_ext[:, D:D + H], approx=True)
    o_all = o_all_ext[:, 0:D] * jnp.dot(rec, sel,
                                        preferred_element_type=jnp.float32)
    attn_out = jnp.dot(o_all.astype(jnp.bfloat16),
                       wo_ref[0].astype(jnp.bfloat16),
                       preferred_element_type=jnp.float32)

    h1 = _add_instance_norm(attn_out, x, n1w_ref[0], n1b_ref[0],
                            batch=batch, seq=seq, eps=eps)

    h1b = h1.astype(jnp.bfloat16)
    w1b = w1_ref[0].astype(jnp.bfloat16)
    Fh = w1b.shape[1] // 2
    fs = []
    for ci in range(2):
        fc = jnp.dot(h1b, w1b[:, ci * Fh:(ci + 1) * Fh],
                     preferred_element_type=jnp.float32).astype(jnp.bfloat16) \
             + b1_ref[0, :, ci * Fh:(ci + 1) * Fh].astype(jnp.bfloat16)
        fs.append(jnp.maximum(fc, jnp.bfloat16(0.0)))
    f = jnp.concatenate(fs, axis=1)
    ffn_out = jnp.dot(f, w2_ref[0].astype(jnp.bfloat16),
                      preferred_element_type=jnp.float32)
    h2 = _add_instance_norm(ffn_out, h1, n2w_ref[0], n2b_ref[0],
                            batch=batch, seq=seq, eps=eps)

    h_ref[...] = h2


def kernel(depot_feats, node_feats, wqkv, bqkv, wo, bo, w1, b1, w2, b2,
           depot_w, node_w, n1_w, n1_b, n2_w, n2_b):
    B, _, Fd = depot_feats.shape
    _, Nc, Fn = node_feats.shape
    D = depot_w.shape[1]
    N = Nc + 1
    M = B * N
    L = wqkv.shape[0]
    H = 8
    eps = 1e-5

    depot_pad = jnp.pad(depot_feats, ((0, 0), (0, 0), (0, Fn)))
    node_pad = jnp.pad(node_feats, ((0, 0), (0, 0), (Fd, 0)))
    feats = jnp.concatenate([depot_pad, node_pad], axis=1).reshape(M, Fd + Fn)
    wcomb = jnp.concatenate([depot_w, node_w], axis=0)

    F = w1.shape[2]

    def full2d(shape):
        return pl.BlockSpec(shape, lambda l: (0, 0))

    def per_layer(shape):
        return pl.BlockSpec((1,) + shape, lambda l: (l, 0, 0))

    body = partial(_encoder_kernel, batch=B, seq=N, num_heads=H, eps=eps)
    init_h, h_out = pl.pallas_call(
        body,
        out_shape=(jax.ShapeDtypeStruct((M, D), jnp.float32),
                   jax.ShapeDtypeStruct((M, D), jnp.float32)),
        grid=(L,),
        in_specs=[
            full2d((M, Fd + Fn)),
            full2d((Fd + Fn, D)),
            per_layer((D, 3 * D)), per_layer((1, 3 * D)),
            per_layer((D, D)), per_layer((1, D)),
            per_layer((D, F)), per_layer((1, F)),
            per_layer((F, D)), per_layer((1, D)),
            per_layer((1, D)), per_layer((1, D)),
            per_layer((1, D)), per_layer((1, D)),
        ],
        out_specs=(full2d((M, D)), full2d((M, D))),
        scratch_shapes=[pltpu.VMEM((8, H * N, D), jnp.bfloat16),
                        pltpu.VMEM((8, H * N, D + H), jnp.bfloat16)],
        compiler_params=pltpu.CompilerParams(
            dimension_semantics=("arbitrary",)),
    )(feats, wcomb,
      wqkv, bqkv, wo, bo,
      w1, b1, w2, b2,
      n1_w, n1_b, n2_w, n2_b)

    return h_out.reshape(B, N, D), init_h.reshape(B, N, D)


# speedup vs baseline: 1.2385x; 1.0480x over previous
import math
from functools import partial

import jax
import jax.numpy as jnp
from jax.experimental import pallas as pl
from jax.experimental.pallas import tpu as pltpu


def _add_instance_norm(x, res, w, b, *, batch, seq, eps):
    d = x.shape[-1]
    h = (x + res).reshape(batch, seq, d)
    mean = jnp.mean(h, axis=1, keepdims=True)
    c = h - mean
    var = jnp.mean(c * c, axis=1, keepdims=True)
    hn = c * jax.lax.rsqrt(var + eps)
    out = hn * w.reshape(1, 1, d) + b.reshape(1, 1, d)
    return out.reshape(batch * seq, d)


def _encoder_kernel(feats_ref, wcomb_ref,
                    wqkv_ref, bqkv_ref, wo_ref, bo_ref,
                    w1_ref, b1_ref, w2_ref, b2_ref,
                    n1w_ref, n1b_ref, n2w_ref, n2b_ref,
                    init_ref, h_ref, kbd_ref, vbd_ref,
                    *, batch, seq, num_heads, eps):
    @pl.when(pl.program_id(0) == 0)
    def _():
        ih = jnp.dot(feats_ref[...], wcomb_ref[...],
                     preferred_element_type=jnp.float32)
        init_ref[...] = ih
        h_ref[...] = ih
        kbd_ref[...] = jnp.zeros_like(kbd_ref)
        nb, rows, _ = vbd_ref.shape
        ii = jax.lax.broadcasted_iota(jnp.int32, (nb, rows, num_heads), 1)
        jj = jax.lax.broadcasted_iota(jnp.int32, (nb, rows, num_heads), 2)
        vbd_ref[:, :, 0:0 + vbd_ref.shape[2]] = jnp.zeros_like(vbd_ref)
        vbd_ref[:, :, kbd_ref.shape[1]:] = \
            jnp.where(ii // seq == jj, 1.0, 0.0).astype(vbd_ref.dtype)

    nbuf = kbd_ref.shape[0]

    _, D = h_ref.shape
    H = num_heads
    hd = D // H
    scale = 1.0 / math.sqrt(hd)
    nt = (((1,), (1,)), ((), ()))

    x = h_ref[...]

    xb = x.astype(jnp.bfloat16)
    wqkv_b = wqkv_ref[0].astype(jnp.bfloat16)

    qs = jnp.dot(xb, wqkv_b[:, 0:D],
                 preferred_element_type=jnp.float32).astype(jnp.bfloat16) \
         + bqkv_ref[0, :, 0:D].astype(jnp.bfloat16)
    qkv = jnp.dot(xb, wqkv_b[:, D:3 * D], preferred_element_type=jnp.float32)

    hh_i = jax.lax.broadcasted_iota(jnp.int32, (H, D), 0)
    cc_i = jax.lax.broadcasted_iota(jnp.int32, (H, D), 1)
    sel = jnp.where(cc_i // hd == hh_i, 1.0, 0.0)

    o_rows = []
    for bi in range(batch):
        r0 = bi * seq
        pb = bi % nbuf
        for hh in range(H):
            c = hh * hd
            kbd_ref[pb, c:c + hd, hh * seq:(hh + 1) * seq] = \
                qkv[r0:r0 + seq, c:c + hd].astype(jnp.bfloat16).T
            vbd_ref[pb, hh * seq:(hh + 1) * seq, c:c + hd] = \
                qkv[r0:r0 + seq, D + c:D + c + hd].astype(jnp.bfloat16)
        q = qs[r0:r0 + seq, :]
        s_cat = jnp.dot(q, kbd_ref[pb], preferred_element_type=jnp.float32)
        p_cat = jnp.exp2(jnp.minimum(
            s_cat * (scale * 1.4426950408889634), 100.0)).astype(jnp.bfloat16)
        o_ext = jnp.dot(p_cat, vbd_ref[pb],
                        preferred_element_type=jnp.float32)
        o_rows.append(o_ext)
    o_all_ext = jnp.concatenate(o_rows, axis=0)
    rec = pl.reciprocal(o_all_ext[:, D:D + H], approx=True)
    o_all = o_all_ext[:, 0:D] * jnp.dot(rec, sel,
                                        preferred_element_type=jnp.float32)
    attn_out = jnp.dot(o_all.astype(jnp.bfloat16),
                       wo_ref[0].astype(jnp.bfloat16),
                       preferred_element_type=jnp.float32)

    h1 = _add_instance_norm(attn_out, x, n1w_ref[0], n1b_ref[0],
                            batch=batch, seq=seq, eps=eps)

    h1b = h1.astype(jnp.bfloat16)
    w1b = w1_ref[0].astype(jnp.bfloat16)
    Fh = w1b.shape[1] // 2
    fs = []
    for ci in range(2):
        fc = jnp.dot(h1b, w1b[:, ci * Fh:(ci + 1) * Fh],
                     preferred_element_type=jnp.float32).astype(jnp.bfloat16) \
             + b1_ref[0, :, ci * Fh:(ci + 1) * Fh].astype(jnp.bfloat16)
        fs.append(jnp.maximum(fc, jnp.bfloat16(0.0)))
    f = jnp.concatenate(fs, axis=1)
    ffn_out = jnp.dot(f, w2_ref[0].astype(jnp.bfloat16),
                      preferred_element_type=jnp.float32)
    h2 = _add_instance_norm(ffn_out, h1, n2w_ref[0], n2b_ref[0],
                            batch=batch, seq=seq, eps=eps)

    h_ref[...] = h2


def kernel(depot_feats, node_feats, wqkv, bqkv, wo, bo, w1, b1, w2, b2,
           depot_w, node_w, n1_w, n1_b, n2_w, n2_b):
    B, _, Fd = depot_feats.shape
    _, Nc, Fn = node_feats.shape
    D = depot_w.shape[1]
    N = Nc + 1
    M = B * N
    L = wqkv.shape[0]
    H = 8
    eps = 1e-5

    depot_pad = jnp.pad(depot_feats, ((0, 0), (0, 0), (0, Fn)))
    node_pad = jnp.pad(node_feats, ((0, 0), (0, 0), (Fd, 0)))
    feats = jnp.concatenate([depot_pad, node_pad], axis=1).reshape(M, Fd + Fn)
    wcomb = jnp.concatenate([depot_w, node_w], axis=0)

    F = w1.shape[2]

    def full2d(shape):
        return pl.BlockSpec(shape, lambda l: (0, 0))

    def per_layer(shape):
        return pl.BlockSpec((1,) + shape, lambda l: (l, 0, 0))

    body = partial(_encoder_kernel, batch=B, seq=N, num_heads=H, eps=eps)
    init_h, h_out = pl.pallas_call(
        body,
        out_shape=(jax.ShapeDtypeStruct((M, D), jnp.float32),
                   jax.ShapeDtypeStruct((M, D), jnp.float32)),
        grid=(L,),
        in_specs=[
            full2d((M, Fd + Fn)),
            full2d((Fd + Fn, D)),
            per_layer((D, 3 * D)), per_layer((1, 3 * D)),
            per_layer((D, D)), per_layer((1, D)),
            per_layer((D, F)), per_layer((1, F)),
            per_layer((F, D)), per_layer((1, D)),
            per_layer((1, D)), per_layer((1, D)),
            per_layer((1, D)), per_layer((1, D)),
        ],
        out_specs=(full2d((M, D)), full2d((M, D))),
        scratch_shapes=[pltpu.VMEM((8, D, H * N), jnp.bfloat16),
                        pltpu.VMEM((8, H * N, D + H), jnp.bfloat16)],
        compiler_params=pltpu.CompilerParams(
            dimension_semantics=("arbitrary",)),
    )(feats, wcomb,
      wqkv, bqkv, wo, bo,
      w1, b1, w2, b2,
      n1_w, n1_b, n2_w, n2_b)

    return h_out.reshape(B, N, D), init_h.reshape(B, N, D)


# speedup vs baseline: 1.2814x; 1.0346x over previous
import math
from functools import partial

import jax
import jax.numpy as jnp
from jax.experimental import pallas as pl
from jax.experimental.pallas import tpu as pltpu


def _add_instance_norm(x, res, w, b, *, batch, seq, eps):
    d = x.shape[-1]
    h = (x + res).reshape(batch, seq, d)
    mean = jnp.mean(h, axis=1, keepdims=True)
    c = h - mean
    var = jnp.mean(c * c, axis=1, keepdims=True)
    hn = c * jax.lax.rsqrt(var + eps)
    out = hn * w.reshape(1, 1, d) + b.reshape(1, 1, d)
    return out.reshape(batch * seq, d)


def _encoder_kernel(feats_ref, wcomb_ref,
                    wqkv_ref, bqkv_ref, wo_ref, bo_ref,
                    w1_ref, b1_ref, w2_ref, b2_ref,
                    n1w_ref, n1b_ref, n2w_ref, n2b_ref,
                    init_ref, h_ref, kbd_ref, vbd_ref,
                    *, batch, seq, num_heads, eps):
    @pl.when(pl.program_id(0) == 0)
    def _():
        ih = jnp.dot(feats_ref[...], wcomb_ref[...],
                     preferred_element_type=jnp.float32)
        init_ref[...] = ih
        h_ref[...] = ih
        kbd_ref[...] = jnp.zeros_like(kbd_ref)
        nb, rows, _ = vbd_ref.shape
        ii = jax.lax.broadcasted_iota(jnp.int32, (nb, rows, num_heads), 1)
        jj = jax.lax.broadcasted_iota(jnp.int32, (nb, rows, num_heads), 2)
        vbd_ref[:, :, 0:0 + vbd_ref.shape[2]] = jnp.zeros_like(vbd_ref)
        vbd_ref[:, :, kbd_ref.shape[1]:] = \
            jnp.where(ii // seq == jj, 1.0, 0.0).astype(vbd_ref.dtype)

    nbuf = kbd_ref.shape[0]

    _, D = h_ref.shape
    H = num_heads
    hd = D // H
    scale = 1.0 / math.sqrt(hd)
    nt = (((1,), (1,)), ((), ()))

    x = h_ref[...]

    xb = x.astype(jnp.bfloat16)
    wqkv_b = wqkv_ref[0].astype(jnp.bfloat16)

    qs = jnp.dot(xb, wqkv_b[:, 0:D],
                 preferred_element_type=jnp.float32).astype(jnp.bfloat16) \
         + bqkv_ref[0, :, 0:D].astype(jnp.bfloat16)
    xT = xb.T
    kt = jax.lax.dot_general(wqkv_b[:, D:2 * D], xT, (((0,), (0,)), ((), ())),
                             preferred_element_type=jnp.float32
                             ).astype(jnp.bfloat16)
    vv = jnp.dot(xb, wqkv_b[:, 2 * D:3 * D],
                 preferred_element_type=jnp.float32)

    hh_i = jax.lax.broadcasted_iota(jnp.int32, (H, D), 0)
    cc_i = jax.lax.broadcasted_iota(jnp.int32, (H, D), 1)
    sel = jnp.where(cc_i // hd == hh_i, 1.0, 0.0)

    o_rows = []
    for bi in range(batch):
        r0 = bi * seq
        pb = bi % nbuf
        for hh in range(H):
            c = hh * hd
            kbd_ref[pb, c:c + hd, hh * seq:(hh + 1) * seq] = \
                kt[c:c + hd, r0:r0 + seq]
            vbd_ref[pb, hh * seq:(hh + 1) * seq, c:c + hd] = \
                vv[r0:r0 + seq, c:c + hd].astype(jnp.bfloat16)
        q = qs[r0:r0 + seq, :]
        s_cat = jnp.dot(q, kbd_ref[pb], preferred_element_type=jnp.float32)
        p_cat = jnp.exp2(jnp.minimum(
            s_cat * (scale * 1.4426950408889634), 100.0)).astype(jnp.bfloat16)
        o_ext = jnp.dot(p_cat, vbd_ref[pb],
                        preferred_element_type=jnp.float32)
        o_rows.append(o_ext)
    o_all_ext = jnp.concatenate(o_rows, axis=0)
    rec = pl.reciprocal(o_all_ext[:, D:D + H], approx=True)
    o_all = o_all_ext[:, 0:D] * jnp.dot(rec, sel,
                                        preferred_element_type=jnp.float32)
    attn_out = jnp.dot(o_all.astype(jnp.bfloat16),
                       wo_ref[0].astype(jnp.bfloat16),
                       preferred_element_type=jnp.float32)

    h1 = _add_instance_norm(attn_out, x, n1w_ref[0], n1b_ref[0],
                            batch=batch, seq=seq, eps=eps)

    h1b = h1.astype(jnp.bfloat16)
    w1b = w1_ref[0].astype(jnp.bfloat16)
    Fh = w1b.shape[1] // 2
    fs = []
    for ci in range(2):
        fc = jnp.dot(h1b, w1b[:, ci * Fh:(ci + 1) * Fh],
                     preferred_element_type=jnp.float32).astype(jnp.bfloat16) \
             + b1_ref[0, :, ci * Fh:(ci + 1) * Fh].astype(jnp.bfloat16)
        fs.append(jnp.maximum(fc, jnp.bfloat16(0.0)))
    f = jnp.concatenate(fs, axis=1)
    ffn_out = jnp.dot(f, w2_ref[0].astype(jnp.bfloat16),
                      preferred_element_type=jnp.float32)
    h2 = _add_instance_norm(ffn_out, h1, n2w_ref[0], n2b_ref[0],
                            batch=batch, seq=seq, eps=eps)

    h_ref[...] = h2


def kernel(depot_feats, node_feats, wqkv, bqkv, wo, bo, w1, b1, w2, b2,
           depot_w, node_w, n1_w, n1_b, n2_w, n2_b):
    B, _, Fd = depot_feats.shape
    _, Nc, Fn = node_feats.shape
    D = depot_w.shape[1]
    N = Nc + 1
    M = B * N
    L = wqkv.shape[0]
    H = 8
    eps = 1e-5

    depot_pad = jnp.pad(depot_feats, ((0, 0), (0, 0), (0, Fn)))
    node_pad = jnp.pad(node_feats, ((0, 0), (0, 0), (Fd, 0)))
    feats = jnp.concatenate([depot_pad, node_pad], axis=1).reshape(M, Fd + Fn)
    wcomb = jnp.concatenate([depot_w, node_w], axis=0)

    F = w1.shape[2]

    def full2d(shape):
        return pl.BlockSpec(shape, lambda l: (0, 0))

    def per_layer(shape):
        return pl.BlockSpec((1,) + shape, lambda l: (l, 0, 0))

    body = partial(_encoder_kernel, batch=B, seq=N, num_heads=H, eps=eps)
    init_h, h_out = pl.pallas_call(
        body,
        out_shape=(jax.ShapeDtypeStruct((M, D), jnp.float32),
                   jax.ShapeDtypeStruct((M, D), jnp.float32)),
        grid=(L,),
        in_specs=[
            full2d((M, Fd + Fn)),
            full2d((Fd + Fn, D)),
            per_layer((D, 3 * D)), per_layer((1, 3 * D)),
            per_layer((D, D)), per_layer((1, D)),
            per_layer((D, F)), per_layer((1, F)),
            per_layer((F, D)), per_layer((1, D)),
            per_layer((1, D)), per_layer((1, D)),
            per_layer((1, D)), per_layer((1, D)),
        ],
        out_specs=(full2d((M, D)), full2d((M, D))),
        scratch_shapes=[pltpu.VMEM((8, D, H * N), jnp.bfloat16),
                        pltpu.VMEM((8, H * N, D + H), jnp.bfloat16)],
        compiler_params=pltpu.CompilerParams(
            dimension_semantics=("arbitrary",)),
    )(feats, wcomb,
      wqkv, bqkv, wo, bo,
      w1, b1, w2, b2,
      n1_w, n1_b, n2_w, n2_b)

    return h_out.reshape(B, N, D), init_h.reshape(B, N, D)
